# baseline (speedup 1.0000x reference)
import functools

import jax
import jax.numpy as jnp
from jax.experimental import pallas as pl
from jax.experimental.pallas import tpu as pltpu


def _gate_kernel(x_ref, w_ref,
                 combine_ref, dispatch_ref, mask1_ref, idx_ref,
                 counts_ref, laux_ref,
                 base_ref, gsum_ref):
    i = pl.program_id(0)
    n = pl.num_programs(0)
    T = x_ref.shape[0]
    E = w_ref.shape[1]
    C = combine_ref.shape[2]

    @pl.when(i == 0)
    def _init():
        base_ref[...] = jnp.zeros_like(base_ref)
        gsum_ref[...] = jnp.zeros_like(gsum_ref)

    x = x_ref[...]
    w = w_ref[...]
    logits = jnp.dot(x, w, preferred_element_type=jnp.float32)

    lmax = jnp.max(logits, axis=1, keepdims=True)
    ex = jnp.exp(logits - lmax)
    gates = ex / jnp.sum(ex, axis=1, keepdims=True)

    gmax = jnp.max(gates, axis=1, keepdims=True)
    eiota = jax.lax.broadcasted_iota(jnp.int32, (T, E), 1)
    idx = jnp.min(jnp.where(gates == gmax, eiota, E), axis=1, keepdims=True)

    mask = (eiota == idx).astype(jnp.int32)
    mask_f = mask.astype(jnp.float32)
    row = jax.lax.broadcasted_iota(jnp.int32, (T, T), 0)
    col = jax.lax.broadcasted_iota(jnp.int32, (T, T), 1)
    tri = (col <= row).astype(jnp.float32)
    csum = jnp.dot(tri, mask_f, preferred_element_type=jnp.float32)
    base = base_ref[...].astype(jnp.float32)
    locs = csum - 1.0 + base
    loc = jnp.sum(locs * mask_f, axis=1, keepdims=True).astype(jnp.int32)

    new_base = base_ref[...] + jnp.sum(mask, axis=0, keepdims=True)
    base_ref[...] = new_base
    gsum_ref[...] = gsum_ref[...] + jnp.sum(gates, axis=0, keepdims=True)

    gate_val = jnp.where(eiota == idx, gmax, 0.0)
    ciota = jax.lax.broadcasted_iota(jnp.int32, (T, C), 1)
    slot = (ciota == loc).astype(jnp.float32)

    combine = gate_val[:, :, None] * slot[:, None, :]
    combine_ref[...] = combine
    dispatch_ref[...] = combine != 0.0

    mask1_ref[...] = mask
    idx_ref[...] = idx

    counts_ref[...] = new_base
    S = n * T
    laux_ref[...] = jnp.sum(
        (gsum_ref[...] / S) * (new_base.astype(jnp.float32) / S),
        keepdims=True,
    ) * E


@functools.partial(jax.jit, static_argnames=("block_t",))
def _top1_gate(x, W, block_t=256):
    S, D = x.shape
    E = W.shape[1]
    import numpy as np
    C = max(int(np.ceil(S / E * 1.0)), 4)
    n = S // block_t

    out_shapes = (
        jax.ShapeDtypeStruct((S, E, C), jnp.float32),
        jax.ShapeDtypeStruct((S, E, C), jnp.bool_),
        jax.ShapeDtypeStruct((S, E), jnp.int32),
        jax.ShapeDtypeStruct((S, 1), jnp.int32),
        jax.ShapeDtypeStruct((1, E), jnp.int32),
        jax.ShapeDtypeStruct((1, 1), jnp.float32),
    )
    return pl.pallas_call(
        _gate_kernel,
        grid=(n,),
        in_specs=[
            pl.BlockSpec((block_t, D), lambda i: (i, 0)),
            pl.BlockSpec((D, E), lambda i: (0, 0)),
        ],
        out_specs=(
            pl.BlockSpec((block_t, E, C), lambda i: (i, 0, 0)),
            pl.BlockSpec((block_t, E, C), lambda i: (i, 0, 0)),
            pl.BlockSpec((block_t, E), lambda i: (i, 0)),
            pl.BlockSpec((block_t, 1), lambda i: (i, 0)),
            pl.BlockSpec((1, E), lambda i: (0, 0)),
            pl.BlockSpec((1, 1), lambda i: (0, 0)),
        ),
        out_shape=out_shapes,
        scratch_shapes=[
            pltpu.VMEM((1, E), jnp.int32),
            pltpu.VMEM((1, E), jnp.float32),
        ],
    )(x.astype(jnp.float32), W)


def kernel(input, W):
    combine, dispatch, mask1, idx, counts, laux = _top1_gate(input, W)
    return (laux[0, 0], combine, dispatch, mask1,
            counts[0], idx[:, 0])


# speedup vs baseline: 1.2620x; 1.2620x over previous
import functools

import jax
import jax.numpy as jnp
from jax.experimental import pallas as pl
from jax.experimental.pallas import tpu as pltpu


def _gate_kernel(x_ref, w_ref,
                 combine_ref, dispatch_ref, mask1_ref, idx_ref,
                 counts_ref, laux_ref,
                 base_ref, gsum_ref):
    i = pl.program_id(0)
    n = pl.num_programs(0)
    T = x_ref.shape[0]
    E = w_ref.shape[1]
    C = combine_ref.shape[2]

    @pl.when(i == 0)
    def _init():
        base_ref[...] = jnp.zeros_like(base_ref)
        gsum_ref[...] = jnp.zeros_like(gsum_ref)

    x = x_ref[...]
    w = w_ref[...]
    logits = jnp.dot(x, w, preferred_element_type=jnp.float32)

    lmax = jnp.max(logits, axis=1, keepdims=True)
    ex = jnp.exp(logits - lmax)
    gates = ex / jnp.sum(ex, axis=1, keepdims=True)

    gmax = jnp.max(gates, axis=1, keepdims=True)
    eiota = jax.lax.broadcasted_iota(jnp.int32, (T, E), 1)
    idx = jnp.min(jnp.where(gates == gmax, eiota, E), axis=1, keepdims=True)

    mask = (eiota == idx).astype(jnp.int32)
    mask_f = mask.astype(jnp.float32)
    row = jax.lax.broadcasted_iota(jnp.int32, (T, T), 0)
    col = jax.lax.broadcasted_iota(jnp.int32, (T, T), 1)
    tri = (col <= row).astype(jnp.float32)
    csum = jnp.dot(tri, mask_f, preferred_element_type=jnp.float32)
    base = base_ref[...].astype(jnp.float32)
    locs = csum - 1.0 + base
    loc = jnp.sum(locs * mask_f, axis=1, keepdims=True).astype(jnp.int32)

    new_base = base_ref[...] + jnp.sum(mask, axis=0, keepdims=True)
    base_ref[...] = new_base
    gsum_ref[...] = gsum_ref[...] + jnp.sum(gates, axis=0, keepdims=True)

    gate_val = jnp.where(eiota == idx, gmax, 0.0)
    ciota = jax.lax.broadcasted_iota(jnp.int32, (T, C), 1)
    slot = (ciota == loc).astype(jnp.float32)

    combine = gate_val[:, :, None] * slot[:, None, :]
    combine_ref[...] = combine
    dispatch_ref[...] = (combine != 0.0).astype(jnp.int8)

    mask1_ref[...] = mask
    idx_ref[...] = idx

    counts_ref[...] = new_base
    S = n * T
    laux_ref[...] = jnp.sum(
        (gsum_ref[...] / S) * (new_base.astype(jnp.float32) / S),
        keepdims=True,
    ) * E


@functools.partial(jax.jit, static_argnames=("block_t",))
def _top1_gate(x, W, block_t=256):
    S, D = x.shape
    E = W.shape[1]
    import numpy as np
    C = max(int(np.ceil(S / E * 1.0)), 4)
    n = S // block_t

    out_shapes = (
        jax.ShapeDtypeStruct((S, E, C), jnp.float32),
        jax.ShapeDtypeStruct((S, E, C), jnp.int8),
        jax.ShapeDtypeStruct((S, E), jnp.int32),
        jax.ShapeDtypeStruct((S, 1), jnp.int32),
        jax.ShapeDtypeStruct((1, E), jnp.int32),
        jax.ShapeDtypeStruct((1, 1), jnp.float32),
    )
    return pl.pallas_call(
        _gate_kernel,
        grid=(n,),
        in_specs=[
            pl.BlockSpec((block_t, D), lambda i: (i, 0)),
            pl.BlockSpec((D, E), lambda i: (0, 0)),
        ],
        out_specs=(
            pl.BlockSpec((block_t, E, C), lambda i: (i, 0, 0)),
            pl.BlockSpec((block_t, E, C), lambda i: (i, 0, 0)),
            pl.BlockSpec((block_t, E), lambda i: (i, 0)),
            pl.BlockSpec((block_t, 1), lambda i: (i, 0)),
            pl.BlockSpec((1, E), lambda i: (0, 0)),
            pl.BlockSpec((1, 1), lambda i: (0, 0)),
        ),
        out_shape=out_shapes,
        scratch_shapes=[
            pltpu.VMEM((1, E), jnp.int32),
            pltpu.VMEM((1, E), jnp.float32),
        ],
    )(x.astype(jnp.float32), W)


def kernel(input, W):
    combine, dispatch, mask1, idx, counts, laux = _top1_gate(input, W)
    return (laux[0, 0], combine, dispatch.astype(jnp.bool_), mask1,
            counts[0], idx[:, 0])
